# baseline (speedup 1.0000x reference)
import functools

import jax
import jax.numpy as jnp
from jax import lax
from jax.experimental import pallas as pl
from jax.experimental.pallas import tpu as pltpu
from jax.experimental.pallas import tpu_sc as plsc

_N = 100000
_NPAD = 100096
_E = 1600000
_G = 64
_GRP = 80
_MB = 5
_NMACRO = _E // (32 * _MB * _GRP)
_NPT = _NPAD // 16
_ZR = 184
_BLK = 4000
_NBLK = _N // _BLK


def _spmm_body(M, table, srcr, dstr, ewr, out, *refs):
    agg, zbuf = refs[0:2]
    sb = refs[2:4]
    db = refs[4:6]
    eb = refs[6:8]
    rows = refs[8:10]
    sem_g = refs[10:12]
    sem_s = refs[12:14]

    c = lax.axis_index("c")
    s = lax.axis_index("s")
    tile = c * 16 + s
    node0 = s * _NPT

    def _zb(i, carry):
        zbuf[i, :] = jnp.zeros((16,), jnp.float32)
        return carry
    lax.fori_loop(0, _ZR, _zb, 0)

    def _load_idx(m, q, k):
        pltpu.sync_copy(srcr.at[tile, m], sb[q])
        pltpu.sync_copy(dstr.at[tile, m], db[q])
        pltpu.sync_copy(ewr.at[tile, m], eb[q])
        if M > 1 or k > 0:
            def _adj(j, carry):
                def _a16(e16, carry2):
                    v = sb[q][j, 0, pl.ds(e16 * 16, 16)]
                    sb[q][j, 0, pl.ds(e16 * 16, 16)] = v * M + k
                    return carry2
                lax.fori_loop(0, _GRP // 16, _a16, 0)
                return carry
            lax.fori_loop(0, _MB, _adj, 0)

    def _fire_gathers(q):
        def _f(j, carry):
            pltpu.async_copy(table.at[sb[q].at[j, 0]], rows[q].at[j],
                             sem_g[q])
            return carry
        lax.fori_loop(0, _MB, _f, 0)

    def _drain_gathers(q):
        def _f(j, carry):
            pltpu.make_async_copy(table.at[sb[q].at[j, 0]],
                                  rows[q].at[j], sem_g[q]).wait()
            return carry
        lax.fori_loop(0, _MB, _f, 0)

    def _process(q):
        def _f(j, carry):
            def _mul(e16, carry2):
                w16 = eb[q][j, 0, pl.ds(e16 * 16, 16)]
                for t in range(16):
                    e = e16 * 16 + t
                    rows[q][j, e, :] = rows[q][j, e, :] * w16[t]
                return carry2
            lax.fori_loop(0, _GRP // 16, _mul, 0)
            pltpu.async_copy(rows[q].at[j], agg.at[db[q].at[j, 0]],
                             sem_s[q], add=True)
            return carry
        lax.fori_loop(0, _MB, _f, 0)

    def _drain_scatters(q):
        def _f(j, carry):
            pltpu.make_async_copy(rows[q].at[j], agg.at[db[q].at[j, 0]],
                                  sem_s[q]).wait()
            return carry
        lax.fori_loop(0, _MB, _f, 0)

    for k in range(M):
        def _zero(i, carry):
            pltpu.sync_copy(zbuf, agg.at[pl.ds(node0 + i * _ZR, _ZR)])
            return carry
        lax.fori_loop(0, _NPT // _ZR, _zero, 0)
        plsc.subcore_barrier()

        _load_idx(0, 0, k)
        _fire_gathers(0)

        def _macro(m, carry):
            for q in range(2):
                @pl.when(m % 2 == q)
                def _body(q=q):
                    nxt = 1 - q

                    @pl.when(m >= 1)
                    def _pre0():
                        _drain_scatters(nxt)

                    @pl.when(m + 1 < _NMACRO)
                    def _pre():
                        _load_idx(m + 1, nxt, k)
                        _fire_gathers(nxt)

                    _drain_gathers(q)
                    _process(q)
            return carry
        lax.fori_loop(0, _NMACRO, _macro, 0)
        _drain_scatters((_NMACRO - 1) % 2)
        plsc.subcore_barrier()
        pltpu.sync_copy(agg.at[pl.ds(node0, _NPT)],
                        out.at[c, pl.ds(node0, _NPT), pl.ds(16 * k, 16)])


@functools.lru_cache(maxsize=None)
def _make_spmm(M):
    mesh = plsc.VectorSubcoreMesh(core_axis_name="c", subcore_axis_name="s")
    out_type = jax.ShapeDtypeStruct((2, _NPAD, 16 * M), jnp.float32)
    scratch = (
        [pltpu.VMEM_SHARED((_NPAD, 16), jnp.float32),
         pltpu.VMEM((_ZR, 16), jnp.float32)]
        + [pltpu.VMEM((_MB, 1, _GRP), jnp.int32) for _ in range(2)]
        + [pltpu.VMEM((_MB, 1, _GRP), jnp.int32) for _ in range(2)]
        + [pltpu.VMEM((_MB, 1, _GRP), jnp.float32) for _ in range(2)]
        + [pltpu.VMEM((_MB, _GRP, 16), jnp.float32) for _ in range(2)]
        + [pltpu.SemaphoreType.DMA for _ in range(4)]
    )
    return pl.kernel(functools.partial(_spmm_body, M), out_type=out_type,
                     mesh=mesh, scratch_types=scratch,
                     compiler_params=pltpu.CompilerParams(use_tc_tiling_on_sc=False))


def _spmm(h_dense, M, srcr, dstr, ewr):
    table = h_dense.reshape(_N * M, 16)
    return _make_spmm(M)(table, srcr, dstr, ewr)


def _combine_body(cout, y_cout, agg_at_out, S, H, *refs):
    i = 0
    if not agg_at_out:
        Wr = refs[i][...]; i += 1
    Wroot = refs[i][...]; i += 1
    b = refs[i][...]; i += 1
    if y_cout:
        Wrel_n = refs[i][...]; i += 1
    outs = refs[i:]

    a_cat = S[0] + S[1]
    if agg_at_out:
        a = a_cat
    else:
        a = jnp.dot(a_cat, Wr, preferred_element_type=jnp.float32)
    r = jnp.dot(H[...], Wroot, preferred_element_type=jnp.float32)
    h = jnp.maximum(a + r + b, 0.0)
    outs[0][...] = h
    if y_cout:
        outs[1][...] = jnp.dot(h, Wrel_n, preferred_element_type=jnp.float32)


@functools.lru_cache(maxsize=None)
def _make_combine(cin_agg, cin, cout, y_cout, agg_at_out):
    body = functools.partial(_combine_body, cout, y_cout, agg_at_out)
    in_specs = [
        pl.BlockSpec((2, _BLK, cin_agg), lambda i: (0, i, 0)),
        pl.BlockSpec((_BLK, cin), lambda i: (i, 0)),
    ]
    if not agg_at_out:
        in_specs.append(pl.BlockSpec((cin_agg, cout), lambda i: (0, 0)))
    in_specs.append(pl.BlockSpec((cin, cout), lambda i: (0, 0)))
    in_specs.append(pl.BlockSpec((1, cout), lambda i: (0, 0)))
    if y_cout:
        in_specs.append(pl.BlockSpec((cout, y_cout), lambda i: (0, 0)))
    out_specs = [pl.BlockSpec((_BLK, cout), lambda i: (i, 0))]
    out_shape = [jax.ShapeDtypeStruct((_N, cout), jnp.float32)]
    if y_cout:
        out_specs.append(pl.BlockSpec((_BLK, y_cout), lambda i: (i, 0)))
        out_shape.append(jax.ShapeDtypeStruct((_N, y_cout), jnp.float32))
    return pl.pallas_call(
        body, grid=(_NBLK,),
        in_specs=in_specs, out_specs=out_specs, out_shape=out_shape)


def _pool_body(S, H, *refs):
    Wroot = refs[0][...]
    b = refs[1][...]
    batch = refs[2]
    w0 = refs[3][...]; b0 = refs[4][...]
    w1 = refs[5][...]; b1 = refs[6][...]
    w2 = refs[7][...]; b2 = refs[8][...]
    out = refs[9]
    acc = refs[10]

    a = S[0] + S[1]
    r = jnp.dot(H[...], Wroot, preferred_element_type=jnp.float32)
    h = jnp.maximum(a + r + b, 0.0)
    hx = jnp.concatenate([h, jnp.ones((_BLK, 16), jnp.float32)], axis=1)
    lab = batch[0]
    iota = lax.broadcasted_iota(jnp.int32, (_G, _BLK), 0)
    mask = (iota == lab).astype(jnp.float32)
    part = jnp.dot(mask, hx, preferred_element_type=jnp.float32)

    g = pl.program_id(0)

    @pl.when(g == 0)
    def _init():
        acc[...] = part

    @pl.when(g > 0)
    def _accum():
        acc[...] = acc[...] + part

    @pl.when(g == _NBLK - 1)
    def _fin():
        tot = acc[...]
        pooled = tot[:, :32] / jnp.maximum(tot[:, 32:33], 1.0)
        z = jnp.maximum(jnp.dot(pooled, w0, preferred_element_type=jnp.float32) + b0, 0.0)
        z = jnp.maximum(jnp.dot(z, w1, preferred_element_type=jnp.float32) + b1, 0.0)
        out[...] = jnp.dot(z, w2, preferred_element_type=jnp.float32) + b2


@functools.lru_cache(maxsize=None)
def _make_pool(cin):
    in_specs = [
        pl.BlockSpec((2, _BLK, 32), lambda i: (0, i, 0)),
        pl.BlockSpec((_BLK, cin), lambda i: (i, 0)),
        pl.BlockSpec((cin, 32), lambda i: (0, 0)),
        pl.BlockSpec((1, 32), lambda i: (0, 0)),
        pl.BlockSpec((1, 1, _BLK), lambda i: (i, 0, 0)),
        pl.BlockSpec((32, 32), lambda i: (0, 0)),
        pl.BlockSpec((1, 32), lambda i: (0, 0)),
        pl.BlockSpec((32, 16), lambda i: (0, 0)),
        pl.BlockSpec((1, 16), lambda i: (0, 0)),
        pl.BlockSpec((16, 1), lambda i: (0, 0)),
        pl.BlockSpec((1, 1), lambda i: (0, 0)),
    ]
    return pl.pallas_call(
        _pool_body, grid=(_NBLK,),
        in_specs=in_specs,
        out_specs=pl.BlockSpec((_G, 1), lambda i: (0, 0)),
        out_shape=jax.ShapeDtypeStruct((_G, 1), jnp.float32),
        scratch_shapes=[pltpu.VMEM((_G, 48), jnp.float32)])


def kernel(x, edge_index, edge_attr, batch,
           W_rel0, b_rel0, W_root0, W_rel1, b_rel1, W_root1,
           W_rel2, b_rel2, W_root2, W_rel3, b_rel3, W_root3,
           W_rel4, b_rel4, W_root4,
           W_mlp0, b_mlp0, W_mlp1, b_mlp1, W_mlp2, b_mlp2):
    srcr = edge_index[0].reshape(32, _NMACRO, _MB, 1, _GRP)
    dstr = edge_index[1].reshape(32, _NMACRO, _MB, 1, _GRP)
    ewr = edge_attr.reshape(32, _NMACRO, _MB, 1, _GRP)
    batchr = batch.reshape(_NBLK, 1, _BLK)

    x_pad = jnp.pad(x, ((0, 0), (0, 11)))
    Wr0 = jnp.pad(W_rel0, ((0, 11), (0, 0)))
    Wrt0 = jnp.pad(W_root0, ((0, 11), (0, 0)))

    S0 = _spmm(x_pad, 1, srcr, dstr, ewr)
    h1, = _make_combine(16, 16, 32, 0, False)(
        S0, x_pad, Wr0, Wrt0, b_rel0.reshape(1, -1))

    S1 = _spmm(h1, 2, srcr, dstr, ewr)
    h2, = _make_combine(32, 32, 64, 0, False)(
        S1, h1, W_rel1, W_root1, b_rel1.reshape(1, -1))

    S2 = _spmm(h2, 4, srcr, dstr, ewr)
    h3, y3 = _make_combine(64, 64, 128, 64, False)(
        S2, h2, W_rel2, W_root2, b_rel2.reshape(1, -1), W_rel3)

    S3 = _spmm(y3, 4, srcr, dstr, ewr)
    h4, y4 = _make_combine(64, 128, 64, 32, True)(
        S3, h3, W_root3, b_rel3.reshape(1, -1), W_rel4)

    S4 = _spmm(y4, 2, srcr, dstr, ewr)
    return _make_pool(64)(
        S4, h4, W_root4, b_rel4.reshape(1, -1), batchr,
        W_mlp0, b_mlp0.reshape(1, -1), W_mlp1, b_mlp1.reshape(1, -1),
        W_mlp2, b_mlp2.reshape(1, -1))
